# Initial kernel scaffold; baseline (speedup 1.0000x reference)
import jax
import jax.numpy as jnp
from jax.experimental import pallas as pl


def kernel(x, edge_index, batch, edge_attr, params):
    raise NotImplementedError("write your pallas kernel here")


# speedup vs baseline: 2.4799x; 2.4799x over previous
import functools

import jax
import jax.numpy as jnp
from jax import lax
from jax.experimental import pallas as pl
from jax.experimental.pallas import tpu as pltpu
from jax.experimental.pallas import tpu_sc as plsc

N = 10000
E = 320000
DX = 144
PW = 32
NC, NS = 2, 16
NW = NC * NS
EW = E // NW
C = 80
NCH = EW // C
NPAIR = (NCH - 1) // 2
RPS = N // NS

BE = 2000
GE = E // BE
BN = 2000
GN = N // BN

f32 = jnp.float32
i32 = jnp.int32


def _sc_mesh():
    return plsc.VectorSubcoreMesh(
        core_axis_name="c", subcore_axis_name="s", num_cores=NC, num_subcores=NS)


@functools.cache
def _make_gather():
    return functools.partial(
        pl.kernel,
        out_type=(jax.ShapeDtypeStruct((E, DX), f32),
                  jax.ShapeDtypeStruct((E, DX), f32)),
        mesh=_sc_mesh(),
        scratch_types=[
            pltpu.VMEM((EW,), i32),
            pltpu.VMEM((EW,), i32),
            pltpu.VMEM((C, DX), f32),
            pltpu.VMEM((C, DX), f32),
            pltpu.VMEM((C, DX), f32),
            pltpu.VMEM((C, DX), f32),
            pltpu.SemaphoreType.DMA,
            pltpu.SemaphoreType.DMA,
            pltpu.SemaphoreType.DMA,
            pltpu.SemaphoreType.DMA,
            pltpu.SemaphoreType.DMA,
            pltpu.SemaphoreType.DMA,
            pltpu.SemaphoreType.DMA,
            pltpu.SemaphoreType.DMA,
        ],
        compiler_params=pltpu.CompilerParams(use_tc_tiling_on_sc=False),
    )(_gather_body)


def _gather_body(x_hbm, src_hbm, dst_hbm, gs_hbm, gd_hbm,
                 sidx, didx, sb0, sb1, db0, db1,
                 gsem0, gsem1, gdsem0, gdsem1, ssem0, ssem1, sdsem0, sdsem1):
    wid = lax.axis_index("s") * NC + lax.axis_index("c")
    base = pl.multiple_of(wid * EW, 8)
    pltpu.sync_copy(src_hbm.at[pl.ds(base, EW)], sidx)
    pltpu.sync_copy(dst_hbm.at[pl.ds(base, EW)], didx)

    sbufs = (sb0, sb1)
    dbufs = (db0, db1)
    gsems = (gsem0, gsem1)
    gdsems = (gdsem0, gdsem1)
    ssems = (ssem0, ssem1)
    sdsems = (sdsem0, sdsem1)

    def issue_gather(ci, b):
        i0 = pl.multiple_of(ci * C, 8)
        pltpu.async_copy(x_hbm.at[sidx.at[pl.ds(i0, C)]], sbufs[b], gsems[b])
        pltpu.async_copy(x_hbm.at[didx.at[pl.ds(i0, C)]], dbufs[b], gdsems[b])

    def wait_gather(b):
        pltpu.make_async_copy(x_hbm.at[sidx.at[pl.ds(0, C)]], sbufs[b], gsems[b]).wait()
        pltpu.make_async_copy(x_hbm.at[didx.at[pl.ds(0, C)]], dbufs[b], gdsems[b]).wait()

    def issue_store(ci, b):
        off = pl.multiple_of(base + ci * C, 8)
        pltpu.async_copy(sbufs[b], gs_hbm.at[pl.ds(off, C)], ssems[b])
        pltpu.async_copy(dbufs[b], gd_hbm.at[pl.ds(off, C)], sdsems[b])

    def wait_store(b):
        pltpu.make_async_copy(sbufs[b], gs_hbm.at[pl.ds(0, C)], ssems[b]).wait()
        pltpu.make_async_copy(dbufs[b], gd_hbm.at[pl.ds(0, C)], sdsems[b]).wait()

    issue_gather(0, 0)
    issue_gather(1, 1)

    def round_body(r, carry):
        for b in (0, 1):
            ci = r * 2 + b
            wait_gather(b)
            issue_store(ci, b)
            wait_store(b)

            @pl.when(ci + 2 < NCH)
            def _():
                issue_gather(ci + 2, b)
        return carry

    lax.fori_loop(0, NPAIR, round_body, 0)
    wait_gather(0)
    issue_store(NCH - 1, 0)
    wait_store(0)


@functools.cache
def _make_scatter():
    return functools.partial(
        pl.kernel,
        out_type=(jax.ShapeDtypeStruct((N, PW), f32),
                  jax.ShapeDtypeStruct((N, PW), f32)),
        mesh=_sc_mesh(),
        scratch_types=[
            pltpu.VMEM((NCH, C), i32),
            pltpu.VMEM((C, PW), f32),
            pltpu.VMEM((C, PW), f32),
            pltpu.VMEM((RPS, PW), f32),
            pltpu.VMEM_SHARED((N, PW), f32),
            pltpu.SemaphoreType.DMA,
            pltpu.SemaphoreType.DMA,
            pltpu.SemaphoreType.DMA,
            pltpu.SemaphoreType.DMA,
        ],
        compiler_params=pltpu.CompilerParams(use_tc_tiling_on_sc=False),
    )(_scatter_body)


def _scatter_body(pay_hbm, dst3_hbm, zero_hbm, out0_hbm, out1_hbm,
                  idx2, pb0, pb1, rowbuf, table,
                  psem0, psem1, asem0, asem1):
    cid = lax.axis_index("c")
    sid = lax.axis_index("s")
    wid = sid * NC + cid
    r0 = sid * RPS

    pltpu.sync_copy(zero_hbm, rowbuf)
    pltpu.sync_copy(rowbuf, table.at[pl.ds(r0, RPS)])
    pltpu.sync_copy(dst3_hbm.at[wid], idx2)
    plsc.subcore_barrier()

    base = wid * EW
    pbufs = (pb0, pb1)
    psems = (psem0, psem1)
    asems = (asem0, asem1)

    def issue_load(ci, b):
        off = pl.multiple_of(base + ci * C, 8)
        pltpu.async_copy(pay_hbm.at[pl.ds(off, C)], pbufs[b], psems[b])

    def wait_load(b):
        pltpu.make_async_copy(pay_hbm.at[pl.ds(0, C)], pbufs[b], psems[b]).wait()

    def issue_add(ci, b):
        pltpu.async_copy(pbufs[b], table.at[idx2.at[ci]], asems[b], add=True)

    def wait_add(b):
        pltpu.make_async_copy(pbufs[b], table.at[idx2.at[0]], asems[b]).wait()

    issue_load(0, 0)
    issue_load(1, 1)

    def round_body(r, carry):
        for b in (0, 1):
            ci = r * 2 + b
            wait_load(b)
            issue_add(ci, b)
            wait_add(b)

            @pl.when(ci + 2 < NCH)
            def _():
                issue_load(ci + 2, b)
        return carry

    lax.fori_loop(0, NPAIR, round_body, 0)
    wait_load(0)
    issue_add(NCH - 1, 0)
    wait_add(0)

    plsc.subcore_barrier()
    pltpu.sync_copy(table.at[pl.ds(r0, RPS)], rowbuf)

    @pl.when(cid == 0)
    def _():
        pltpu.sync_copy(rowbuf, out0_hbm.at[pl.ds(r0, RPS)])

    @pl.when(cid == 1)
    def _():
        pltpu.sync_copy(rowbuf, out1_hbm.at[pl.ds(r0, RPS)])


def _edge_body(gs_ref, gd_ref, ea_ref, w1a, w1b, w1c, w1d, b1,
               w2, b2, cw1, cb1, cw2, cb2, out_ref):
    gs = gs_ref[...]
    gd = gd_ref[...]
    hs = gs[:, :128]
    hd = gd[:, :128]
    rel = gs[:, 128:132] - gd[:, 128:132]
    dist = jnp.sqrt(jnp.sum(rel * rel, axis=1, keepdims=True))
    z1 = (jnp.dot(hd, w1a[...], preferred_element_type=f32)
          + jnp.dot(hs, w1b[...], preferred_element_type=f32)
          + jnp.dot(ea_ref[...], w1c[...], preferred_element_type=f32)
          + dist * w1d[...]
          + b1[...])
    a1 = jax.nn.silu(z1)
    m = jax.nn.silu(jnp.dot(a1, w2[...], preferred_element_type=f32) + b2[...])
    t = jax.nn.silu(jnp.dot(m, cw1[...], preferred_element_type=f32) + cb1[...])
    cw = jnp.dot(t, cw2[...], preferred_element_type=f32) + cb2[...]
    out_ref[...] = jnp.concatenate(
        [m, cw, rel, jnp.zeros((BE, PW - 21), f32)], axis=1)


_edge_call = pl.pallas_call(
    _edge_body,
    grid=(GE,),
    in_specs=[
        pl.BlockSpec((BE, DX), lambda i: (i, 0)),
        pl.BlockSpec((BE, DX), lambda i: (i, 0)),
        pl.BlockSpec((BE, 16), lambda i: (i, 0)),
        pl.BlockSpec((128, 546), lambda i: (0, 0)),
        pl.BlockSpec((128, 546), lambda i: (0, 0)),
        pl.BlockSpec((16, 546), lambda i: (0, 0)),
        pl.BlockSpec((1, 546), lambda i: (0, 0)),
        pl.BlockSpec((1, 546), lambda i: (0, 0)),
        pl.BlockSpec((546, 16), lambda i: (0, 0)),
        pl.BlockSpec((1, 16), lambda i: (0, 0)),
        pl.BlockSpec((16, 64), lambda i: (0, 0)),
        pl.BlockSpec((1, 64), lambda i: (0, 0)),
        pl.BlockSpec((64, 1), lambda i: (0, 0)),
        pl.BlockSpec((1, 1), lambda i: (0, 0)),
    ],
    out_specs=pl.BlockSpec((BE, PW), lambda i: (i, 0)),
    out_shape=jax.ShapeDtypeStruct((E, PW), f32),
)


def _node_body(x_ref, p0_ref, p1_ref, nw1a, nw1b, nb1, nw2, nb2, out_ref):
    x = x_ref[...]
    h = x[:, :128]
    p = p0_ref[...] + p1_ref[...]
    m_i = p[:, :16]
    z = (jnp.dot(h, nw1a[...], preferred_element_type=f32)
         + jnp.dot(m_i, nw1b[...], preferred_element_type=f32)
         + nb1[...])
    hidden = jnp.dot(jax.nn.silu(z), nw2[...], preferred_element_type=f32) + nb2[...] + h
    cnew = x[:, 128:132] + p[:, 16:17] * p[:, 17:21]
    out_ref[...] = jnp.concatenate(
        [hidden, cnew, jnp.zeros((BN, DX - 132), f32)], axis=1)


_node_call = pl.pallas_call(
    _node_body,
    grid=(GN,),
    in_specs=[
        pl.BlockSpec((BN, DX), lambda i: (i, 0)),
        pl.BlockSpec((BN, PW), lambda i: (i, 0)),
        pl.BlockSpec((BN, PW), lambda i: (i, 0)),
        pl.BlockSpec((128, 256), lambda i: (0, 0)),
        pl.BlockSpec((16, 256), lambda i: (0, 0)),
        pl.BlockSpec((1, 256), lambda i: (0, 0)),
        pl.BlockSpec((256, 128), lambda i: (0, 0)),
        pl.BlockSpec((1, 128), lambda i: (0, 0)),
    ],
    out_specs=pl.BlockSpec((BN, DX), lambda i: (i, 0)),
    out_shape=jax.ShapeDtypeStruct((N, DX), f32),
)


def kernel(x, edge_index, batch, edge_attr, params):
    del batch
    src = edge_index[0]
    dst = edge_index[1]
    dst3 = dst.reshape(NW, NCH, C)
    x144 = jnp.concatenate([x, jnp.zeros((N, DX - 131), f32)], axis=1)
    zeros_blk = jnp.zeros((RPS, PW), f32)
    gather_fn = _make_gather()
    scatter_fn = _make_scatter()
    for p in params:
        gs, gd = gather_fn(x144, src, dst)
        payload = _edge_call(
            gs, gd, edge_attr,
            p['ew1'][:128], p['ew1'][128:256], p['ew1'][256:272],
            p['ew1'][272:273], p['eb1'][None, :],
            p['ew2'], p['eb2'][None, :],
            p['cw1'], p['cb1'][None, :], p['cw2'], p['cb2'][None, :])
        pt0, pt1 = scatter_fn(payload, dst3, zeros_blk)
        x144 = _node_call(
            x144, pt0, pt1,
            p['nw1'][:128], p['nw1'][128:144], p['nb1'][None, :],
            p['nw2'], p['nb2'][None, :])
    return x144[:, :131]
